# Initial kernel scaffold; baseline (speedup 1.0000x reference)
import jax
import jax.numpy as jnp
from jax.experimental import pallas as pl


def kernel(input_ids, attention_mask, question_time, embed_table, Wq, Wv, kb_keys, kb_values, kb_ctx, tau_min, tau_max, w_ctx, w_gate):
    raise NotImplementedError("write your pallas kernel here")


# speedup vs baseline: 2.0102x; 2.0102x over previous
import functools

import jax
import jax.numpy as jnp
from jax import lax
from jax.experimental import pallas as pl
from jax.experimental.pallas import tpu as pltpu
from jax.experimental.pallas import tpu_sc as plsc

B, T = 8, 4
D_MODEL = 1024
D_K = 64
D_V = 64
D_CTX = 16
K_TOP = 32
KB_N = 1000000
VOCAB = 32000
BT = B * T

NEG = -3.0e38

SC_NC = 2
SC_NS = 16
SC_NW = SC_NC * SC_NS

_EMB_WORKERS = 4
_EMB_ROWS = BT // _EMB_WORKERS


def _sc_embed_gather(table, ids):
    mesh = plsc.VectorSubcoreMesh(core_axis_name="c", subcore_axis_name="s")

    @functools.partial(
        pl.kernel,
        mesh=mesh,
        out_type=jax.ShapeDtypeStruct((BT, D_MODEL), jnp.float32),
        scratch_types=[
            pltpu.VMEM((_EMB_ROWS,), jnp.int32),
            pltpu.VMEM((_EMB_ROWS, D_MODEL), jnp.float32),
            pltpu.SemaphoreType.DMA,
        ],
    )
    def k(table_hbm, idx_hbm, out_hbm, idx_v, rows_v, sem):
        wid = lax.axis_index("s") * SC_NC + lax.axis_index("c")

        @pl.when(wid < _EMB_WORKERS)
        def _():
            base = wid * _EMB_ROWS
            pltpu.sync_copy(idx_hbm.at[pl.ds(base, _EMB_ROWS)], idx_v)
            pltpu.async_copy(table_hbm.at[idx_v], rows_v, sem).wait()
            pltpu.sync_copy(rows_v, out_hbm.at[pl.ds(base, _EMB_ROWS)])

    return k(table, ids)


BLK = 8192
NBLK = -(-KB_N // BLK)
NBUK = 4096
DEPTH = 4
SUBS = BLK // NBUK


def _topk_body(hidden_ref, wq_ref, keys_ref, q_ref, idx_ref, val_ref,
               vals, idxs):
    step = pl.program_id(0)

    @pl.when(step == 0)
    def _():
        q_ref[...] = jnp.dot(
            hidden_ref[...], wq_ref[...], preferred_element_type=jnp.float32
        )
        vals[...] = jnp.full((DEPTH, BT, NBUK), NEG, jnp.float32)
        idxs[...] = jnp.zeros((DEPTH, BT, NBUK), jnp.int32)

    s = lax.dot_general(
        q_ref[...],
        keys_ref[...],
        (((1,), (1,)), ((), ())),
        preferred_element_type=jnp.float32,
    )

    base = step * BLK
    for sub in range(SUBS):
        v = s[:, sub * NBUK:(sub + 1) * NBUK]
        gi = (base + sub * NBUK) + lax.broadcasted_iota(jnp.int32, (BT, NBUK), 1)
        v = jnp.where(gi < KB_N, v, NEG)
        for l in range(DEPTH):
            rv = vals[l]
            ri = idxs[l]
            swap = v > rv
            nv = jnp.where(swap, v, rv)
            ni = jnp.where(swap, gi, ri)
            v = jnp.where(swap, rv, v)
            gi = jnp.where(swap, ri, gi)
            vals[l] = nv
            idxs[l] = ni

    @pl.when(step == NBLK - 1)
    def _():
        ti = idxs[...]
        col = lax.broadcasted_iota(jnp.int32, (BT, K_TOP), 1)

        def extract(j, carry):
            tv, oi, ov = carry
            m = jnp.max(jnp.max(tv, axis=0), axis=1)
            sel = tv == m[None, :, None]
            cand = jnp.where(sel, ti, jnp.int32(2147483647))
            pick = jnp.min(jnp.min(cand, axis=0), axis=1)
            oi = jnp.where(col == j, pick[:, None], oi)
            ov = jnp.where(col == j, m[:, None], ov)
            tv = jnp.where(sel & (ti == pick[None, :, None]), NEG, tv)
            return tv, oi, ov

        _, oi, ov = lax.fori_loop(
            0, K_TOP, extract,
            (vals[...],
             jnp.zeros((BT, K_TOP), jnp.int32),
             jnp.zeros((BT, K_TOP), jnp.float32)))
        idx_ref[...] = oi
        val_ref[...] = ov


def _tc_score_topk(hidden, wq, kb_keys):
    return pl.pallas_call(
        _topk_body,
        grid=(NBLK,),
        in_specs=[
            pl.BlockSpec((BT, D_MODEL), lambda i: (0, 0)),
            pl.BlockSpec((D_MODEL, D_K), lambda i: (0, 0)),
            pl.BlockSpec((BLK, D_K), lambda i: (i, 0)),
        ],
        out_specs=[
            pl.BlockSpec((BT, D_K), lambda i: (0, 0)),
            pl.BlockSpec((BT, K_TOP), lambda i: (0, 0)),
            pl.BlockSpec((BT, K_TOP), lambda i: (0, 0)),
        ],
        out_shape=[
            jax.ShapeDtypeStruct((BT, D_K), jnp.float32),
            jax.ShapeDtypeStruct((BT, K_TOP), jnp.int32),
            jax.ShapeDtypeStruct((BT, K_TOP), jnp.float32),
        ],
        scratch_shapes=[
            pltpu.VMEM((DEPTH, BT, NBUK), jnp.float32),
            pltpu.VMEM((DEPTH, BT, NBUK), jnp.int32),
        ],
    )(hidden, wq, kb_keys)


VB = 3200
NVB = VOCAB // VB
NFETCH = BT * K_TOP


def _finish_body(idxs_ref, hidden_ref, sv_ref, idx_ref, idxc_ref,
                 qmin_ref, qmax_ref,
                 am_ref, wv_ref, wc_ref, wg_ref, emb_ref,
                 values_any, ctx_any, tmin_any, tmax_any,
                 out_ref, h2_ref, vbuf, cbuf, t0buf, t1buf, sem):
    step = pl.program_id(0)

    @pl.when(step == 0)
    def _():
        def row_copies(j):
            i = idxs_ref[j]
            i8 = pl.multiple_of((i // 8) * 8, 8)
            i128 = pl.multiple_of((i // 128) * 128, 128)
            return [
                pltpu.make_async_copy(
                    values_any.at[pl.ds(i8, 8), :], vbuf.at[j], sem),
                pltpu.make_async_copy(
                    ctx_any.at[pl.ds(i8, 8), :], cbuf.at[j], sem),
                pltpu.make_async_copy(
                    tmin_any.at[pl.ds(i128, 128)], t0buf.at[j, 0], sem),
                pltpu.make_async_copy(
                    tmax_any.at[pl.ds(i128, 128)], t1buf.at[j, 0], sem),
            ]

        def fire(j, carry):
            for c in row_copies(j):
                c.start()
            return carry

        lax.fori_loop(0, NFETCH, fire, 0)

        def drain(j, carry):
            for c in row_copies(j):
                c.wait()
            return carry

        lax.fori_loop(0, NFETCH, drain, 0)

        att = sv_ref[...] * jnp.float32(0.125)
        sub8 = idxc_ref[...] % 8
        row8 = lax.broadcasted_iota(jnp.int32, (NFETCH, 8), 1)
        oh8 = (row8 == sub8).astype(jnp.float32)[:, :, None]
        cc = jnp.sum(cbuf[...] * oh8, axis=1).reshape(BT, K_TOP, D_CTX)
        att = att + jnp.sum(cc * wc_ref[...][None, :, :], axis=2)
        sub = idx_ref[...] % 128
        lane = lax.broadcasted_iota(jnp.int32, (BT, K_TOP, 128), 2)
        onehot = (lane == sub[:, :, None]).astype(jnp.float32)
        tmin = jnp.sum(t0buf[...].reshape(BT, K_TOP, 128) * onehot, axis=2)
        tmax = jnp.sum(t1buf[...].reshape(BT, K_TOP, 128) * onehot, axis=2)
        valid = (tmin <= qmax_ref[...]) & (tmax >= qmin_ref[...])
        att = jnp.where(valid, att, jnp.float32(-1e9))
        att = att - jnp.max(att, axis=1, keepdims=True)
        e = jnp.exp(att)
        alpha = e / jnp.sum(e, axis=1, keepdims=True)
        vv = jnp.sum(vbuf[...] * oh8, axis=1).reshape(BT, K_TOP, D_V)
        vt = jnp.sum(vv * alpha[:, :, None], axis=1)
        vproj = jnp.dot(vt, wv_ref[...], preferred_element_type=jnp.float32)
        g = jnp.dot(hidden_ref[...], wg_ref[...],
                    preferred_element_type=jnp.float32)
        beta = (jnp.float32(1.0) / (jnp.float32(1.0) + jnp.exp(-g))) * am_ref[...]
        h2_ref[...] = hidden_ref[...] + beta * vproj

    out_ref[...] = lax.dot_general(
        h2_ref[...],
        emb_ref[...],
        (((1,), (1,)), ((), ())),
        preferred_element_type=jnp.float32,
    )


def _tc_finish(idxs, hidden, sv, idx, qmin, qmax, am, wv, wc, wg, emb,
               kb_values, kb_ctx, tau_min, tau_max):
    return pl.pallas_call(
        _finish_body,
        grid=(NVB,),
        in_specs=[
            pl.BlockSpec(memory_space=pltpu.SMEM),
            pl.BlockSpec((BT, D_MODEL), lambda i: (0, 0)),
            pl.BlockSpec((BT, K_TOP), lambda i: (0, 0)),
            pl.BlockSpec((BT, K_TOP), lambda i: (0, 0)),
            pl.BlockSpec((NFETCH, 1), lambda i: (0, 0)),
            pl.BlockSpec((BT, 1), lambda i: (0, 0)),
            pl.BlockSpec((BT, 1), lambda i: (0, 0)),
            pl.BlockSpec((BT, 1), lambda i: (0, 0)),
            pl.BlockSpec((D_V, D_MODEL), lambda i: (0, 0)),
            pl.BlockSpec((1, D_CTX), lambda i: (0, 0)),
            pl.BlockSpec((D_MODEL, 1), lambda i: (0, 0)),
            pl.BlockSpec((VB, D_MODEL), lambda i: (i, 0)),
            pl.BlockSpec(memory_space=pl.ANY),
            pl.BlockSpec(memory_space=pl.ANY),
            pl.BlockSpec(memory_space=pl.ANY),
            pl.BlockSpec(memory_space=pl.ANY),
        ],
        out_specs=pl.BlockSpec((BT, VB), lambda i: (0, i)),
        out_shape=jax.ShapeDtypeStruct((BT, VOCAB), jnp.float32),
        scratch_shapes=[
            pltpu.VMEM((BT, D_MODEL), jnp.float32),
            pltpu.VMEM((NFETCH, 8, D_V), jnp.float32),
            pltpu.VMEM((NFETCH, 8, D_CTX), jnp.float32),
            pltpu.VMEM((NFETCH, 1, 128), jnp.float32),
            pltpu.VMEM((NFETCH, 1, 128), jnp.float32),
            pltpu.SemaphoreType.DMA,
        ],
    )(idxs, hidden, sv, idx, idxs.reshape(NFETCH, 1), qmin, qmax, am,
      wv, wc, wg, emb, kb_values, kb_ctx, tau_min, tau_max)


def kernel(input_ids, attention_mask, question_time, embed_table, Wq, Wv,
           kb_keys, kb_values, kb_ctx, tau_min, tau_max, w_ctx, w_gate):
    ids = input_ids.reshape(BT).astype(jnp.int32)
    hidden = _sc_embed_gather(embed_table, ids)

    q, idx, sv = _tc_score_topk(hidden, Wq, kb_keys)

    qmin = jnp.repeat(question_time[:, 0:1], T, axis=1).reshape(BT, 1)
    qmax = jnp.repeat(question_time[:, 1:2], T, axis=1).reshape(BT, 1)
    am = attention_mask.reshape(BT, 1)

    tau_min_p = jnp.pad(tau_min, (0, 128))
    tau_max_p = jnp.pad(tau_max, (0, 128))

    logits = _tc_finish(
        idx.reshape(BT * K_TOP), hidden, sv, idx, qmin, qmax, am,
        Wv, w_ctx.reshape(1, D_CTX), w_gate.reshape(D_MODEL, 1), embed_table,
        kb_values, kb_ctx, tau_min_p, tau_max_p,
    )
    return logits.reshape(B, T, VOCAB)
